# Initial kernel scaffold; baseline (speedup 1.0000x reference)
import jax
import jax.numpy as jnp
from jax.experimental import pallas as pl


def kernel(nodes, neigh0, neigh1, embedding, W_self0, W_neigh0, W_self1, W_neigh1, dense):
    raise NotImplementedError("write your pallas kernel here")


# speedup vs baseline: 2.4517x; 2.4517x over previous
import functools

import jax
import jax.numpy as jnp
from jax import lax
from jax.experimental import pallas as pl
from jax.experimental.pallas import tpu as pltpu
from jax.experimental.pallas import tpu_sc as plsc

NW = 32
D = 512
NV = D // 16

F0, C0 = 25, 4
F1, C1 = 10, 8


def _sc_gather_sums(embedding, idx_self, idx0, idx1, B):
    bpw = B // NW
    nch0 = bpw // C0
    nch1 = bpw // C1
    mesh = plsc.VectorSubcoreMesh(core_axis_name="c", subcore_axis_name="s")

    @functools.partial(
        pl.kernel,
        mesh=mesh,
        out_type=(
            jax.ShapeDtypeStruct((B, D), jnp.float32),
            jax.ShapeDtypeStruct((B, D), jnp.float32),
            jax.ShapeDtypeStruct((B, D), jnp.float32),
        ),
        scratch_types=[
            pltpu.VMEM((bpw,), jnp.int32),
            pltpu.VMEM((nch0, C0 * F0), jnp.int32),
            pltpu.VMEM((nch1, C1 * F1), jnp.int32),
            pltpu.VMEM((C0 * F0, D), jnp.float32),
            pltpu.VMEM((bpw, D), jnp.float32),
            pltpu.SemaphoreType.DMA,
        ],
    )
    def k(emb_hbm, idxs_hbm, idx0_hbm, idx1_hbm,
          self_hbm, s0_hbm, s1_hbm,
          idxs_v, idx0_v, idx1_v, rows_v, out_v, sem):
        wid = lax.axis_index("s") * 2 + lax.axis_index("c")
        base = wid * bpw

        pltpu.sync_copy(idxs_hbm.at[wid], idxs_v)
        pltpu.async_copy(emb_hbm.at[idxs_v], out_v, sem).wait()
        pltpu.sync_copy(out_v, self_hbm.at[pl.ds(base, bpw)])

        def neigh_phase(idx_hbm, idx_v, nch, C, F, dst_hbm):
            pltpu.sync_copy(idx_hbm.at[wid], idx_v)

            def chunk_body(ci, _):
                pltpu.async_copy(
                    emb_hbm.at[idx_v.at[ci]], rows_v.at[pl.ds(0, C * F)], sem
                ).wait()
                for n in range(C):
                    def row_body(j, accs):
                        return tuple(
                            accs[d] + rows_v[n * F + j, pl.ds(d * 16, 16)]
                            for d in range(NV)
                        )
                    accs = tuple(
                        rows_v[n * F, pl.ds(d * 16, 16)] for d in range(NV)
                    )
                    accs = lax.fori_loop(1, F, row_body, accs)
                    for d in range(NV):
                        out_v[ci * C + n, pl.ds(d * 16, 16)] = accs[d]
                return 0

            lax.fori_loop(0, nch, chunk_body, 0)
            pltpu.sync_copy(out_v, dst_hbm.at[pl.ds(base, bpw)])

        neigh_phase(idx0_hbm, idx0_v, nch0, C0, F0, s0_hbm)
        neigh_phase(idx1_hbm, idx1_v, nch1, C1, F1, s1_hbm)

    return k(embedding, idx_self, idx0, idx1)


def _tc_dense_chain(sv, s0, s1, W_self0, W_neigh0, W_self1, W_neigh1, dense,
                    inv0, inv1):
    B = sv.shape[0]
    BM = 512
    H = W_self0.shape[1]
    L = dense.shape[1]

    def body(sv_ref, s0_ref, s1_ref, ws0, wn0, ws1, wn1, dn, out_ref):
        f32 = jnp.float32
        h = jnp.dot(sv_ref[...], ws0[...], preferred_element_type=f32)
        h += jnp.dot(s0_ref[...] * inv0, wn0[...], preferred_element_type=f32)
        h = jnp.maximum(h, 0.0)
        h2 = jnp.dot(h, ws1[...], preferred_element_type=f32)
        h2 += jnp.dot(s1_ref[...] * inv1, wn1[...], preferred_element_type=f32)
        h2 = jnp.maximum(h2, 0.0)
        out_ref[...] = jax.nn.sigmoid(
            jnp.dot(h2, dn[...], preferred_element_type=f32))

    grid = (B // BM,)
    row_spec = pl.BlockSpec((BM, D), lambda i: (i, 0))
    return pl.pallas_call(
        body,
        grid=grid,
        in_specs=[
            row_spec, row_spec, row_spec,
            pl.BlockSpec((D, H), lambda i: (0, 0)),
            pl.BlockSpec((D, H), lambda i: (0, 0)),
            pl.BlockSpec((H, H), lambda i: (0, 0)),
            pl.BlockSpec((D, H), lambda i: (0, 0)),
            pl.BlockSpec((H, L), lambda i: (0, 0)),
        ],
        out_specs=pl.BlockSpec((BM, L), lambda i: (i, 0)),
        out_shape=jax.ShapeDtypeStruct((B, L), jnp.float32),
    )(sv, s0, s1, W_self0, W_neigh0, W_self1, W_neigh1, dense)


def kernel(nodes, neigh0, neigh1, embedding, W_self0, W_neigh0, W_self1,
           W_neigh1, dense):
    B = nodes.shape[0]
    bpw = B // NW
    idx_self = nodes.astype(jnp.int32).reshape(NW, bpw)
    idx0 = neigh0.astype(jnp.int32).reshape(NW, bpw // C0, C0 * F0)
    idx1 = neigh1.astype(jnp.int32).reshape(NW, bpw // C1, C1 * F1)
    sv, s0, s1 = _sc_gather_sums(embedding, idx_self, idx0, idx1, B)
    return _tc_dense_chain(sv, s0, s1, W_self0, W_neigh0, W_self1, W_neigh1,
                           dense, 1.0 / F0, 1.0 / F1)
